# baseline (speedup 1.0000x reference)
import jax
import jax.numpy as jnp
from jax.experimental import pallas as pl
from jax.experimental.pallas import tpu as pltpu

N = 256
D = 128
NEG = 0.2
RC = 8


def _gat3_body(x_ref, adjT_ref,
               Wl1, bl1, Wr1, br1, att1, bias1,
               Wl2, bl2, Wr2, br2, att2, bias2,
               Wl3, bl3, Wr3, br3, att3, bias3,
               out_ref, yr_s, C_s, x_s):
    maskbias = jnp.where(adjT_ref[...] > 0.0, 0.0, -jnp.inf)
    x = x_ref[...]
    for li, (Wl, bl, Wr, br, att, bias) in enumerate((
        (Wl1, bl1, Wr1, br1, att1, bias1),
        (Wl2, bl2, Wr2, br2, att2, bias2),
        (Wl3, bl3, Wr3, br3, att3, bias3),
    )):
        xl = jnp.dot(x, Wl[...], preferred_element_type=jnp.float32) + bl[...]
        xr = jnp.dot(x, Wr[...], preferred_element_type=jnp.float32) + br[...]
        attr = att[...]
        attabs = jnp.abs(attr)
        sgnv = jnp.sign(attr).reshape(1, 1, D)
        yl = xl * attabs
        yr_s[...] = xr * attabs
        attc = attr.reshape(D, 1)
        sr = jnp.dot(xr, attc, preferred_element_type=jnp.float32)
        slT = jax.lax.dot_general(attr, xl, (((1,), (1,)), ((), ())),
                                  preferred_element_type=jnp.float32)
        C_s[...] = maskbias + 0.6 * sr + 0.6 * slT
        biasv = bias[...]
        dst = out_ref if li == 2 else x_s

        def chunk(i, carry):
            yrc = yr_s[pl.ds(i * RC, RC), :]
            m = yrc[:, None, :] + yl[None, :, :]
            red = jnp.sum(jnp.abs(m) * sgnv, axis=-1)
            Sm = 0.4 * red + C_s[pl.ds(i * RC, RC), :]
            amax = jnp.max(Sm, axis=1, keepdims=True)
            amax = jnp.where(amax == -jnp.inf, 0.0, amax)
            e = jnp.exp(Sm - amax)
            den = jnp.sum(e, axis=1, keepdims=True)
            A = e / (den + 1e-16)
            dst[pl.ds(i * RC, RC), :] = (
                jnp.dot(A, xl, preferred_element_type=jnp.float32) + biasv)
            return carry

        jax.lax.fori_loop(0, N // RC, chunk, 0)
        if li != 2:
            x = x_s[...]


def kernel(batch_graph, adj, Wl1, bl1, Wr1, br1, att1, bias1,
           Wl2, bl2, Wr2, br2, att2, bias2,
           Wl3, bl3, Wr3, br3, att3, bias3):
    B = batch_graph.shape[0]
    adjT = jnp.swapaxes(adj, 1, 2)
    vecs = [v.reshape(1, -1) for v in (bl1, br1, att1, bias1,
                                       bl2, br2, att2, bias2,
                                       bl3, br3, att3, bias3)]
    (bl1, br1, att1, bias1, bl2, br2, att2, bias2,
     bl3, br3, att3, bias3) = vecs
    weights = (Wl1, bl1, Wr1, br1, att1, bias1,
               Wl2, bl2, Wr2, br2, att2, bias2,
               Wl3, bl3, Wr3, br3, att3, bias3)

    def _full(w):
        return pl.BlockSpec(w.shape, lambda b: (0,) * w.ndim)

    out = pl.pallas_call(
        _gat3_body,
        grid=(B,),
        in_specs=[pl.BlockSpec((None, N, D), lambda b: (b, 0, 0)),
                  pl.BlockSpec((None, N, N), lambda b: (b, 0, 0))]
                 + [_full(w) for w in weights],
        out_specs=pl.BlockSpec((None, N, D), lambda b: (b, 0, 0)),
        out_shape=jax.ShapeDtypeStruct((B, N, D), jnp.float32),
        scratch_shapes=[pltpu.VMEM((N, D), jnp.float32),
                        pltpu.VMEM((N, N), jnp.float32),
                        pltpu.VMEM((N, D), jnp.float32)],
    )(batch_graph, adjT, *weights)
    return out


# speedup vs baseline: 5.1148x; 5.1148x over previous
import jax
import jax.numpy as jnp
from jax.experimental import pallas as pl
from jax.experimental.pallas import tpu as pltpu

N = 256
D = 128
NEG = 0.2
RC = 8


def _gat3_body(x_ref, adjT_ref,
               Wl1, bl1, Wr1, br1, att1, bias1,
               Wl2, bl2, Wr2, br2, att2, bias2,
               Wl3, bl3, Wr3, br3, att3, bias3,
               out_ref, yr_s, S_s):
    maskbias = jnp.where(adjT_ref[...] > 0.0, 0.0, -jnp.inf)
    x = x_ref[...]
    for (Wl, bl, Wr, br, att, bias) in (
        (Wl1, bl1, Wr1, br1, att1, bias1),
        (Wl2, bl2, Wr2, br2, att2, bias2),
        (Wl3, bl3, Wr3, br3, att3, bias3),
    ):
        xl = jnp.dot(x, Wl[...], preferred_element_type=jnp.float32) + bl[...]
        xr = jnp.dot(x, Wr[...], preferred_element_type=jnp.float32) + br[...]
        attr = att[...]
        attabs = jnp.abs(attr)
        sgnv = jnp.sign(attr).reshape(1, 1, D)
        yl = xl * attabs
        yr_s[...] = xr * attabs
        attc = attr.reshape(D, 1)
        sr = jnp.dot(xr, attc, preferred_element_type=jnp.float32)
        slT = jax.lax.dot_general(attr, xl, (((1,), (1,)), ((), ())),
                                  preferred_element_type=jnp.float32)

        def chunk(i, carry):
            yrc = yr_s[pl.ds(i * RC, RC), :]
            m = yrc[:, None, :] + yl[None, :, :]
            S_s[pl.ds(i * RC, RC), :] = jnp.sum(jnp.abs(m) * sgnv, axis=-1)
            return carry

        jax.lax.fori_loop(0, N // RC, chunk, 0)

        Sm = 0.4 * S_s[...] + (0.6 * sr + maskbias) + 0.6 * slT
        amax = jnp.max(Sm, axis=1, keepdims=True)
        amax = jnp.where(amax == -jnp.inf, 0.0, amax)
        e = jnp.exp(Sm - amax)
        denom = jnp.sum(e, axis=1, keepdims=True)
        A = e / (denom + 1e-16)
        x = jnp.dot(A, xl, preferred_element_type=jnp.float32) + bias[...]
    out_ref[...] = x


def kernel(batch_graph, adj, Wl1, bl1, Wr1, br1, att1, bias1,
           Wl2, bl2, Wr2, br2, att2, bias2,
           Wl3, bl3, Wr3, br3, att3, bias3):
    B = batch_graph.shape[0]
    adjT = jnp.swapaxes(adj, 1, 2)
    vecs = [v.reshape(1, -1) for v in (bl1, br1, att1, bias1,
                                       bl2, br2, att2, bias2,
                                       bl3, br3, att3, bias3)]
    (bl1, br1, att1, bias1, bl2, br2, att2, bias2,
     bl3, br3, att3, bias3) = vecs
    weights = (Wl1, bl1, Wr1, br1, att1, bias1,
               Wl2, bl2, Wr2, br2, att2, bias2,
               Wl3, bl3, Wr3, br3, att3, bias3)

    def _full(w):
        return pl.BlockSpec(w.shape, lambda b: (0,) * w.ndim)

    out = pl.pallas_call(
        _gat3_body,
        grid=(B,),
        in_specs=[pl.BlockSpec((None, N, D), lambda b: (b, 0, 0)),
                  pl.BlockSpec((None, N, N), lambda b: (b, 0, 0))]
                 + [_full(w) for w in weights],
        out_specs=pl.BlockSpec((None, N, D), lambda b: (b, 0, 0)),
        out_shape=jax.ShapeDtypeStruct((B, N, D), jnp.float32),
        scratch_shapes=[pltpu.VMEM((N, D), jnp.float32),
                        pltpu.VMEM((N, N), jnp.float32)],
    )(batch_graph, adjT, *weights)
    return out


# speedup vs baseline: 5.9039x; 1.1543x over previous
import jax
import jax.numpy as jnp
from jax.experimental import pallas as pl
from jax.experimental.pallas import tpu as pltpu

N = 256
D = 128
RC = 128


def _gat3_body(x_ref, adjT_ref,
               Wl1, bl1, Wr1, br1, att1, bias1,
               Wl2, bl2, Wr2, br2, att2, bias2,
               Wl3, bl3, Wr3, br3, att3, bias3,
               out_ref, yr_s, S_s):
    maskbias = jnp.where(adjT_ref[...] > 0.0, 0.0, -jnp.inf)
    x = x_ref[...]
    for (Wl, bl, Wr, br, att, bias) in (
        (Wl1, bl1, Wr1, br1, att1, bias1),
        (Wl2, bl2, Wr2, br2, att2, bias2),
        (Wl3, bl3, Wr3, br3, att3, bias3),
    ):
        xl = jnp.dot(x, Wl[...], preferred_element_type=jnp.float32) + bl[...]
        xr = jnp.dot(x, Wr[...], preferred_element_type=jnp.float32) + br[...]
        attr = att[...]
        attabs = jnp.abs(attr)
        sgnv = jnp.sign(attr).reshape(1, 1, D)
        yl = xl * attabs
        yr_s[...] = xr * attabs
        attc = attr.reshape(D, 1)
        sr = jnp.dot(xr, attc, preferred_element_type=jnp.float32)
        slT = jax.lax.dot_general(attr, xl, (((1,), (1,)), ((), ())),
                                  preferred_element_type=jnp.float32)

        for i in range(N // RC):
            yrc = yr_s[pl.ds(i * RC, RC), :]
            m = yrc[:, None, :] + yl[None, :, :]
            S_s[pl.ds(i * RC, RC), :] = jnp.sum(jnp.abs(m) * sgnv, axis=-1)

        Sm = 0.4 * S_s[...] + (0.6 * sr + maskbias) + 0.6 * slT
        amax = jnp.max(Sm, axis=1, keepdims=True)
        amax = jnp.where(amax == -jnp.inf, 0.0, amax)
        e = jnp.exp(Sm - amax)
        denom = jnp.sum(e, axis=1, keepdims=True)
        A = e / (denom + 1e-16)
        x = jnp.dot(A, xl, preferred_element_type=jnp.float32) + bias[...]
    out_ref[...] = x


def kernel(batch_graph, adj, Wl1, bl1, Wr1, br1, att1, bias1,
           Wl2, bl2, Wr2, br2, att2, bias2,
           Wl3, bl3, Wr3, br3, att3, bias3):
    B = batch_graph.shape[0]
    adjT = jnp.swapaxes(adj, 1, 2)
    vecs = [v.reshape(1, -1) for v in (bl1, br1, att1, bias1,
                                       bl2, br2, att2, bias2,
                                       bl3, br3, att3, bias3)]
    (bl1, br1, att1, bias1, bl2, br2, att2, bias2,
     bl3, br3, att3, bias3) = vecs
    weights = (Wl1, bl1, Wr1, br1, att1, bias1,
               Wl2, bl2, Wr2, br2, att2, bias2,
               Wl3, bl3, Wr3, br3, att3, bias3)

    def _full(w):
        return pl.BlockSpec(w.shape, lambda b: (0,) * w.ndim)

    out = pl.pallas_call(
        _gat3_body,
        grid=(B,),
        in_specs=[pl.BlockSpec((None, N, D), lambda b: (b, 0, 0)),
                  pl.BlockSpec((None, N, N), lambda b: (b, 0, 0))]
                 + [_full(w) for w in weights],
        out_specs=pl.BlockSpec((None, N, D), lambda b: (b, 0, 0)),
        out_shape=jax.ShapeDtypeStruct((B, N, D), jnp.float32),
        scratch_shapes=[pltpu.VMEM((N, D), jnp.float32),
                        pltpu.VMEM((N, N), jnp.float32)],
    )(batch_graph, adjT, *weights)
    return out
